# Initial kernel scaffold; baseline (speedup 1.0000x reference)
import jax
import jax.numpy as jnp
from jax.experimental import pallas as pl


def kernel(input_ids, weight):
    raise NotImplementedError("write your pallas kernel here")


# speedup vs baseline: 1.5573x; 1.5573x over previous
import functools

import jax
import jax.numpy as jnp
from jax import lax
from jax.experimental import pallas as pl
from jax.experimental.pallas import tpu as pltpu
from jax.experimental.pallas import tpu_sc as plsc

_NC = 2
_NS = 16
_NW = _NC * _NS

_CHUNK = 32


@functools.partial(jax.jit, static_argnames=())
def _sc_embedding_gather(weight, idx3):
    n_chunks = idx3.shape[1]
    d_model = weight.shape[1]
    b_total = _NW * n_chunks * _CHUNK
    b_per_w = n_chunks * _CHUNK

    mesh = plsc.VectorSubcoreMesh(core_axis_name="c", subcore_axis_name="s")

    @functools.partial(
        pl.kernel,
        mesh=mesh,
        out_type=jax.ShapeDtypeStruct((b_total, d_model), jnp.float32),
        scratch_types=[
            pltpu.VMEM((n_chunks, _CHUNK), jnp.int32),
            pltpu.VMEM((_CHUNK, d_model), jnp.float32),
            pltpu.SemaphoreType.DMA,
        ],
    )
    def k(table_hbm, idx_hbm, out_hbm, idx_v, rows_v, sem):
        wid = lax.axis_index("s") * _NC + lax.axis_index("c")
        base = wid * b_per_w
        pltpu.sync_copy(idx_hbm.at[wid], idx_v)
        for c in range(n_chunks):
            pltpu.async_copy(table_hbm.at[idx_v.at[c]], rows_v, sem).wait()
            pltpu.sync_copy(rows_v, out_hbm.at[pl.ds(base + c * _CHUNK, _CHUNK)])

    return k(weight, idx3)


def kernel(input_ids, weight):
    batch, seq = input_ids.shape
    b_total = batch * seq
    n_chunks = b_total // (_NW * _CHUNK)
    idx3 = input_ids.astype(jnp.int32).reshape(_NW, n_chunks, _CHUNK)
    out = _sc_embedding_gather(weight, idx3)
    return out.reshape(batch, seq, weight.shape[1])


# speedup vs baseline: 1.6465x; 1.0573x over previous
import functools

import jax
import jax.numpy as jnp
from jax import lax
from jax.experimental import pallas as pl
from jax.experimental.pallas import tpu as pltpu
from jax.experimental.pallas import tpu_sc as plsc

_NC = 2
_NS = 16
_NW = _NC * _NS

_CHUNK = 16
_NBUF = 2


def _sc_embedding_gather(weight, idx3):
    n_chunks = idx3.shape[1]
    d_model = weight.shape[1]
    b_per_w = n_chunks * _CHUNK
    b_total = _NW * b_per_w

    mesh = plsc.VectorSubcoreMesh(core_axis_name="c", subcore_axis_name="s")

    @functools.partial(
        pl.kernel,
        mesh=mesh,
        out_type=jax.ShapeDtypeStruct((b_total, d_model), jnp.float32),
        scratch_types=[
            pltpu.VMEM((n_chunks, _CHUNK), jnp.int32),
            pltpu.VMEM((_CHUNK, d_model), jnp.float32),
            pltpu.VMEM((_CHUNK, d_model), jnp.float32),
            pltpu.SemaphoreType.DMA,
            pltpu.SemaphoreType.DMA,
            pltpu.SemaphoreType.DMA,
            pltpu.SemaphoreType.DMA,
        ],
    )
    def k(table_hbm, idx_hbm, out_hbm, idx_v, rows0, rows1, g0, g1, w0, w1):
        wid = lax.axis_index("s") * _NC + lax.axis_index("c")
        base = wid * b_per_w
        bufs, gsems, wsems = [rows0, rows1], [g0, g1], [w0, w1]
        pltpu.sync_copy(idx_hbm.at[wid], idx_v)
        gathers, writes = {}, {}
        gathers[0] = pltpu.async_copy(table_hbm.at[idx_v.at[0]], bufs[0],
                                      gsems[0])
        for c in range(n_chunks):
            b = c % _NBUF
            if c + 1 < n_chunks:
                nb = (c + 1) % _NBUF
                if c - 1 >= 0:
                    writes[c - 1].wait()
                gathers[c + 1] = pltpu.async_copy(
                    table_hbm.at[idx_v.at[c + 1]], bufs[nb], gsems[nb])
            gathers[c].wait()
            writes[c] = pltpu.async_copy(
                bufs[b], out_hbm.at[pl.ds(base + c * _CHUNK, _CHUNK)],
                wsems[b])
        writes[n_chunks - 2].wait()
        writes[n_chunks - 1].wait()

    return k(weight, idx3)


def kernel(input_ids, weight):
    batch, seq = input_ids.shape
    b_total = batch * seq
    n_chunks = b_total // (_NW * _CHUNK)
    idx3 = input_ids.astype(jnp.int32).reshape(_NW, n_chunks, _CHUNK)
    out = _sc_embedding_gather(weight, idx3)
    return out.reshape(batch, seq, weight.shape[1])


# speedup vs baseline: 1.6672x; 1.0126x over previous
import functools

import jax
import jax.numpy as jnp
from jax import lax
from jax.experimental import pallas as pl
from jax.experimental.pallas import tpu as pltpu
from jax.experimental.pallas import tpu_sc as plsc

_NC = 2
_NS = 16
_NW = _NC * _NS

_CHUNK = 16
_NBUF = 3


def _sc_embedding_gather(weight, idx3):
    n_chunks = idx3.shape[1]
    d_model = weight.shape[1]
    b_per_w = n_chunks * _CHUNK
    b_total = _NW * b_per_w

    mesh = plsc.VectorSubcoreMesh(core_axis_name="c", subcore_axis_name="s")

    @functools.partial(
        pl.kernel,
        mesh=mesh,
        out_type=jax.ShapeDtypeStruct((b_total, d_model), jnp.float32),
        scratch_types=(
            [pltpu.VMEM((n_chunks, _CHUNK), jnp.int32)]
            + [pltpu.VMEM((_CHUNK, d_model), jnp.float32)] * _NBUF
            + [pltpu.SemaphoreType.DMA] * (2 * _NBUF)
        ),
    )
    def k(table_hbm, idx_hbm, out_hbm, idx_v, *bufs_and_sems):
        bufs = list(bufs_and_sems[:_NBUF])
        gsems = list(bufs_and_sems[_NBUF:2 * _NBUF])
        wsems = list(bufs_and_sems[2 * _NBUF:])
        wid = lax.axis_index("s") * _NC + lax.axis_index("c")
        base = wid * b_per_w
        pltpu.sync_copy(idx_hbm.at[wid], idx_v)
        gathers, writes = {}, {}
        for c in range(min(_NBUF, n_chunks)):
            gathers[c] = pltpu.async_copy(table_hbm.at[idx_v.at[c]],
                                          bufs[c % _NBUF], gsems[c % _NBUF])
        for c in range(n_chunks):
            b = c % _NBUF
            gathers[c].wait()
            writes[c] = pltpu.async_copy(
                bufs[b], out_hbm.at[pl.ds(base + c * _CHUNK, _CHUNK)],
                wsems[b])
            nxt = c + _NBUF
            if nxt < n_chunks:
                writes[nxt - _NBUF].wait()
                gathers[nxt] = pltpu.async_copy(
                    table_hbm.at[idx_v.at[nxt]], bufs[b], gsems[b])
        for c in range(max(0, n_chunks - _NBUF), n_chunks):
            writes[c].wait()

    return k(weight, idx3)


def kernel(input_ids, weight):
    batch, seq = input_ids.shape
    b_total = batch * seq
    n_chunks = b_total // (_NW * _CHUNK)
    idx3 = input_ids.astype(jnp.int32).reshape(_NW, n_chunks, _CHUNK)
    out = _sc_embedding_gather(weight, idx3)
    return out.reshape(batch, seq, weight.shape[1])


# speedup vs baseline: 1.6791x; 1.0071x over previous
import functools

import jax
import jax.numpy as jnp
from jax import lax
from jax.experimental import pallas as pl
from jax.experimental.pallas import tpu as pltpu
from jax.experimental.pallas import tpu_sc as plsc

_NC = 2
_NS = 16
_NW = _NC * _NS

_CHUNK = 16
_NBUF = 3


def _sc_embedding_gather(weight, input_ids):
    batch, seq = input_ids.shape
    d_model = weight.shape[1]
    b_per_w = (batch * seq) // _NW
    n_chunks = b_per_w // _CHUNK
    w_per_row = seq // b_per_w

    mesh = plsc.VectorSubcoreMesh(core_axis_name="c", subcore_axis_name="s")

    @functools.partial(
        pl.kernel,
        mesh=mesh,
        out_type=jax.ShapeDtypeStruct((batch, seq, d_model), jnp.float32),
        scratch_types=(
            [pltpu.VMEM((b_per_w,), jnp.int32)]
            + [pltpu.VMEM((_CHUNK, d_model), jnp.float32)] * _NBUF
            + [pltpu.SemaphoreType.DMA] * (2 * _NBUF)
        ),
    )
    def k(table_hbm, idx_hbm, out_hbm, idx_v, *bufs_and_sems):
        bufs = list(bufs_and_sems[:_NBUF])
        gsems = list(bufs_and_sems[_NBUF:2 * _NBUF])
        wsems = list(bufs_and_sems[2 * _NBUF:])
        wid = lax.axis_index("s") * _NC + lax.axis_index("c")
        row = wid // w_per_row
        col0 = (wid % w_per_row) * b_per_w
        pltpu.sync_copy(idx_hbm.at[row, pl.ds(col0, b_per_w)], idx_v)
        gathers, writes = {}, {}
        for c in range(min(_NBUF, n_chunks)):
            gathers[c] = pltpu.async_copy(
                table_hbm.at[idx_v.at[pl.ds(c * _CHUNK, _CHUNK)]],
                bufs[c % _NBUF], gsems[c % _NBUF])
        for c in range(n_chunks):
            b = c % _NBUF
            gathers[c].wait()
            writes[c] = pltpu.async_copy(
                bufs[b],
                out_hbm.at[row, pl.ds(col0 + c * _CHUNK, _CHUNK)],
                wsems[b])
            nxt = c + _NBUF
            if nxt < n_chunks:
                writes[nxt - _NBUF].wait()
                gathers[nxt] = pltpu.async_copy(
                    table_hbm.at[idx_v.at[pl.ds(nxt * _CHUNK, _CHUNK)]],
                    bufs[b], gsems[b])
        for c in range(max(0, n_chunks - _NBUF), n_chunks):
            writes[c].wait()

    return k(weight, input_ids)


def kernel(input_ids, weight):
    return _sc_embedding_gather(weight, input_ids.astype(jnp.int32))
